# Initial kernel scaffold; baseline (speedup 1.0000x reference)
import functools

import jax
import jax.numpy as jnp
from jax import lax
from jax.experimental import pallas as pl
from jax.experimental.pallas import tpu as pltpu
from jax.experimental.pallas import tpu_sc as plsc

N = 10000
E = 320000
D = 128

NC = 2
NS = 16
NW = NC * NS

EPW = E // NW
CHUNK = 80
NCHUNK = EPW // CHUNK
RPS = N // NS

_mesh = plsc.VectorSubcoreMesh(core_axis_name="c", subcore_axis_name="s")


def _sc_aggregate(x, src, dst, zrows, zdeg, ones_chunk, with_deg):
    out_types = [jax.ShapeDtypeStruct((NC, N, D), jnp.float32)]
    scratch = [
        pltpu.VMEM((CHUNK,), jnp.int32),
        pltpu.VMEM((CHUNK,), jnp.int32),
        pltpu.VMEM((CHUNK, D), jnp.float32),
        pltpu.VMEM_SHARED((N, D), jnp.float32),
        pltpu.SemaphoreType.DMA,
    ]
    if with_deg:
        out_types.append(jax.ShapeDtypeStruct((NC, N, 16), jnp.float32))
        scratch += [
            pltpu.VMEM((CHUNK, 16), jnp.float32),
            pltpu.VMEM_SHARED((N, 16), jnp.float32),
        ]

    def body(*refs):
        if with_deg:
            (x_hbm, src_hbm, dst_hbm, z_hbm, zd_hbm, ones_hbm,
             acc_out, deg_out, sidx, didx, rows, acc_sh, sem,
             ones_v, deg_sh) = refs
        else:
            (x_hbm, src_hbm, dst_hbm, z_hbm, zd_hbm, ones_hbm,
             acc_out, sidx, didx, rows, acc_sh, sem) = refs

        c = lax.axis_index("c")
        s = lax.axis_index("s")
        base = (c * NS + s) * EPW

        pltpu.sync_copy(z_hbm, acc_sh.at[pl.ds(s * RPS, RPS)])
        if with_deg:
            pltpu.sync_copy(zd_hbm, deg_sh.at[pl.ds(s * RPS, RPS)])
            pltpu.sync_copy(ones_hbm, ones_v)
        plsc.subcore_barrier()

        @pl.loop(0, NCHUNK)
        def _(ci):
            off = base + ci * CHUNK
            pltpu.sync_copy(src_hbm.at[pl.ds(off, CHUNK)], sidx)
            pltpu.sync_copy(dst_hbm.at[pl.ds(off, CHUNK)], didx)
            pltpu.async_copy(x_hbm.at[sidx], rows, sem).wait()
            pltpu.sync_copy(rows, acc_sh.at[didx], add=True)
            if with_deg:
                pltpu.sync_copy(ones_v, deg_sh.at[didx], add=True)

        plsc.subcore_barrier()
        sl = pl.ds(s * RPS, RPS)
        pltpu.sync_copy(acc_sh.at[sl], acc_out.at[c].at[sl])
        if with_deg:
            pltpu.sync_copy(deg_sh.at[sl], deg_out.at[c].at[sl])

    run = pl.kernel(body, out_type=tuple(out_types), mesh=_mesh,
                    scratch_types=scratch)
    return run(x, src, dst, zrows, zdeg, ones_chunk)


_BR = 1000


def _tc_combine(acc, deg16, x, W_l, W_r, b, relu):
    def body(acc_ref, deg_ref, x_ref, wl_ref, wr_ref, b_ref, o_ref):
        s = acc_ref[0] + acc_ref[1]
        deg = deg_ref[0, :, 0] + deg_ref[1, :, 0]
        mean = s / jnp.clip(deg, 1.0, None)[:, None]
        y = (jnp.dot(mean, wl_ref[...], preferred_element_type=jnp.float32)
             + jnp.dot(x_ref[...], wr_ref[...], preferred_element_type=jnp.float32)
             + b_ref[...])
        o_ref[...] = jnp.maximum(y, 0.0) if relu else y

    return pl.pallas_call(
        body,
        grid=(N // _BR,),
        in_specs=[
            pl.BlockSpec((NC, _BR, D), lambda i: (0, i, 0)),
            pl.BlockSpec((NC, _BR, 16), lambda i: (0, i, 0)),
            pl.BlockSpec((_BR, D), lambda i: (i, 0)),
            pl.BlockSpec((D, D), lambda i: (0, 0)),
            pl.BlockSpec((D, D), lambda i: (0, 0)),
            pl.BlockSpec((1, D), lambda i: (0, 0)),
        ],
        out_specs=pl.BlockSpec((_BR, D), lambda i: (i, 0)),
        out_shape=jax.ShapeDtypeStruct((N, D), jnp.float32),
    )(acc, deg16, x, W_l, W_r, b)


@jax.jit
def kernel(x, edge_index, W_l1, W_r1, b1, W_l2, W_r2, b2):
    src = edge_index[0].astype(jnp.int32)
    dst = edge_index[1].astype(jnp.int32)
    zrows = jnp.zeros((RPS, D), jnp.float32)
    zdeg = jnp.zeros((RPS, 16), jnp.float32)
    ones_chunk = jnp.ones((CHUNK, 16), jnp.float32)
    b1r = b1.reshape(1, D)
    b2r = b2.reshape(1, D)

    acc1, deg16 = _sc_aggregate(x, src, dst, zrows, zdeg, ones_chunk, True)
    h = _tc_combine(acc1, deg16, x, W_l1, W_r1, b1r, True)
    (acc2,) = _sc_aggregate(h, src, dst, zrows, zdeg, ones_chunk, False)
    out = _tc_combine(acc2, deg16, h, W_l2, W_r2, b2r, False)
    return out


# speedup vs baseline: 4.8263x; 4.8263x over previous
import jax
import jax.numpy as jnp
from jax import lax
from jax.experimental import pallas as pl
from jax.experimental.pallas import tpu as pltpu
from jax.experimental.pallas import tpu_sc as plsc

N = 10000
E = 320000
D = 128

NC = 2
NS = 16
NW = NC * NS

EPW = E // NW
CHUNK = 80
NCHUNK = EPW // CHUNK
NP_ = 10240
RPS = NP_ // NS

_mesh = plsc.VectorSubcoreMesh(core_axis_name="c", subcore_axis_name="s")


def _sc_aggregate(x, ef, zrows):
    def body(x_hbm, ef_hbm, z_hbm, acc_out, sidx, didx, rows, acc_sh, sem):
        c = lax.axis_index("c")
        s = lax.axis_index("s")
        base = (c * NS + s) * EPW

        pltpu.sync_copy(z_hbm, acc_sh.at[pl.ds(s * RPS, RPS)])
        plsc.subcore_barrier()

        @pl.loop(0, NCHUNK)
        def _(ci):
            off = base + ci * CHUNK
            pltpu.sync_copy(ef_hbm.at[pl.ds(off, CHUNK)], sidx)
            pltpu.sync_copy(ef_hbm.at[pl.ds(E + off, CHUNK)], didx)
            pltpu.async_copy(x_hbm.at[sidx], rows, sem).wait()
            pltpu.sync_copy(rows, acc_sh.at[didx], add=True)

        plsc.subcore_barrier()
        pltpu.sync_copy(acc_sh.at[pl.ds(s * RPS, RPS)],
                        acc_out.at[pl.ds(c * NP_ + s * RPS, RPS)])

    run = pl.kernel(body, out_type=jax.ShapeDtypeStruct((NC * NP_, D), jnp.float32),
                    mesh=_mesh,
                    scratch_types=[
                        pltpu.VMEM((CHUNK,), jnp.int32),
                        pltpu.VMEM((CHUNK,), jnp.int32),
                        pltpu.VMEM((CHUNK, D), jnp.float32),
                        pltpu.VMEM_SHARED((NP_, D), jnp.float32),
                        pltpu.SemaphoreType.DMA,
                    ])
    return run(x, ef, zrows)


def _sc_degree(ef, zrows, ones_rows):
    def body(ef_hbm, z_hbm, ones_hbm, deg_out, didx, ones_v, acc_sh):
        c = lax.axis_index("c")
        s = lax.axis_index("s")
        base = (c * NS + s) * EPW

        pltpu.sync_copy(z_hbm, acc_sh.at[pl.ds(s * RPS, RPS)])
        pltpu.sync_copy(ones_hbm, ones_v)
        plsc.subcore_barrier()

        @pl.loop(0, NCHUNK)
        def _(ci):
            off = base + ci * CHUNK
            pltpu.sync_copy(ef_hbm.at[pl.ds(E + off, CHUNK)], didx)
            pltpu.sync_copy(ones_v, acc_sh.at[didx], add=True)

        plsc.subcore_barrier()
        pltpu.sync_copy(acc_sh.at[pl.ds(s * RPS, RPS)],
                        deg_out.at[pl.ds(c * NP_ + s * RPS, RPS)])

    run = pl.kernel(body, out_type=jax.ShapeDtypeStruct((NC * NP_, D), jnp.float32),
                    mesh=_mesh,
                    scratch_types=[
                        pltpu.VMEM((CHUNK,), jnp.int32),
                        pltpu.VMEM((CHUNK, D), jnp.float32),
                        pltpu.VMEM_SHARED((NP_, D), jnp.float32),
                    ])
    return run(ef, zrows, ones_rows)


_BR = 1000


def _tc_combine(acc, degw, x, W_l, W_r, b, relu):
    def body(acc_ref, deg_ref, x_ref, wl_ref, wr_ref, b_ref, o_ref):
        s = acc_ref[0] + acc_ref[1]
        deg = deg_ref[0, :, 0] + deg_ref[1, :, 0]
        mean = s / jnp.clip(deg, 1.0, None)[:, None]
        y = (jnp.dot(mean, wl_ref[...], preferred_element_type=jnp.float32)
             + jnp.dot(x_ref[...], wr_ref[...], preferred_element_type=jnp.float32)
             + b_ref[...])
        o_ref[...] = jnp.maximum(y, 0.0) if relu else y

    return pl.pallas_call(
        body,
        grid=(N // _BR,),
        in_specs=[
            pl.BlockSpec((NC, _BR, D), lambda i: (0, i, 0)),
            pl.BlockSpec((NC, _BR, D), lambda i: (0, i, 0)),
            pl.BlockSpec((_BR, D), lambda i: (i, 0)),
            pl.BlockSpec((D, D), lambda i: (0, 0)),
            pl.BlockSpec((D, D), lambda i: (0, 0)),
            pl.BlockSpec((1, D), lambda i: (0, 0)),
        ],
        out_specs=pl.BlockSpec((_BR, D), lambda i: (i, 0)),
        out_shape=jax.ShapeDtypeStruct((N, D), jnp.float32),
    )(acc, degw, x, W_l, W_r, b)


@jax.jit
def kernel(x, edge_index, W_l1, W_r1, b1, W_l2, W_r2, b2):
    ef = edge_index.astype(jnp.int32).reshape(2 * E)
    zrows = jnp.zeros((RPS, D), jnp.float32)
    ones_rows = jnp.ones((CHUNK, D), jnp.float32)
    b1r = b1.reshape(1, D)
    b2r = b2.reshape(1, D)

    degw = _sc_degree(ef, zrows, ones_rows).reshape(NC, NP_, D)
    acc1 = _sc_aggregate(x, ef, zrows).reshape(NC, NP_, D)
    h = _tc_combine(acc1, degw, x, W_l1, W_r1, b1r, True)
    acc2 = _sc_aggregate(h, ef, zrows).reshape(NC, NP_, D)
    out = _tc_combine(acc2, degw, h, W_l2, W_r2, b2r, False)
    return out


# speedup vs baseline: 7.0732x; 1.4655x over previous
import jax
import jax.numpy as jnp
from jax import lax
from jax.experimental import pallas as pl
from jax.experimental.pallas import tpu as pltpu
from jax.experimental.pallas import tpu_sc as plsc

N = 10000
E = 320000
D = 128

NC = 2
NS = 16
NW = NC * NS

EPW = E // NW
CHUNK = 80
NCHUNK = EPW // CHUNK
NP_ = 10240
RPS = NP_ // NS

_mesh = plsc.VectorSubcoreMesh(core_axis_name="c", subcore_axis_name="s")


def _sc_aggregate(x, ef, zrows):
    def body(x_hbm, ef_hbm, z_hbm, acc_out,
             sa0, sa1, da0, da1, rows0, rows1, acc_sh, sem):
        c = lax.axis_index("c")
        s = lax.axis_index("s")
        base = (c * NS + s) * EPW

        pltpu.sync_copy(z_hbm, acc_sh.at[pl.ds(s * RPS, RPS)])
        plsc.subcore_barrier()

        pltpu.sync_copy(ef_hbm.at[pl.ds(base, CHUNK)], sa0)
        pltpu.sync_copy(ef_hbm.at[pl.ds(E + base, CHUNK)], da0)
        pltpu.async_copy(x_hbm.at[sa0], rows0, sem)

        @pl.loop(0, NCHUNK, step=2)
        def _(ci):
            @pl.when(ci + 1 < NCHUNK)
            def _():
                off = base + (ci + 1) * CHUNK
                pltpu.sync_copy(ef_hbm.at[pl.ds(off, CHUNK)], sa1)
                pltpu.sync_copy(ef_hbm.at[pl.ds(E + off, CHUNK)], da1)

            pltpu.make_async_copy(x_hbm.at[sa0], rows0, sem).wait()

            @pl.when(ci + 1 < NCHUNK)
            def _():
                pltpu.async_copy(x_hbm.at[sa1], rows1, sem)

            pltpu.sync_copy(rows0, acc_sh.at[da0], add=True)

            @pl.when(ci + 2 < NCHUNK)
            def _():
                off = base + (ci + 2) * CHUNK
                pltpu.sync_copy(ef_hbm.at[pl.ds(off, CHUNK)], sa0)
                pltpu.sync_copy(ef_hbm.at[pl.ds(E + off, CHUNK)], da0)

            @pl.when(ci + 1 < NCHUNK)
            def _():
                pltpu.make_async_copy(x_hbm.at[sa1], rows1, sem).wait()

                @pl.when(ci + 2 < NCHUNK)
                def _():
                    pltpu.async_copy(x_hbm.at[sa0], rows0, sem)

                pltpu.sync_copy(rows1, acc_sh.at[da1], add=True)

        plsc.subcore_barrier()
        pltpu.sync_copy(acc_sh.at[pl.ds(s * RPS, RPS)],
                        acc_out.at[pl.ds(c * NP_ + s * RPS, RPS)])

    run = pl.kernel(body, out_type=jax.ShapeDtypeStruct((NC * NP_, D), jnp.float32),
                    mesh=_mesh,
                    scratch_types=[
                        pltpu.VMEM((CHUNK,), jnp.int32),
                        pltpu.VMEM((CHUNK,), jnp.int32),
                        pltpu.VMEM((CHUNK,), jnp.int32),
                        pltpu.VMEM((CHUNK,), jnp.int32),
                        pltpu.VMEM((CHUNK, D), jnp.float32),
                        pltpu.VMEM((CHUNK, D), jnp.float32),
                        pltpu.VMEM_SHARED((NP_, D), jnp.float32),
                        pltpu.SemaphoreType.DMA,
                    ])
    return run(x, ef, zrows)


def _sc_degree(dst, zrows, ones_rows):
    DC = 80
    DNC = EPW // DC

    def body(d_hbm, z_hbm, ones_hbm, deg_out, didx, ones_v, acc_sh, sem):
        c = lax.axis_index("c")
        s = lax.axis_index("s")
        base = (c * NS + s) * EPW

        pltpu.sync_copy(z_hbm, acc_sh.at[pl.ds(s * RPS, RPS)])
        pltpu.sync_copy(ones_hbm, ones_v)
        plsc.subcore_barrier()

        @pl.loop(0, DNC)
        def _(ci):
            off = base + ci * DC
            pltpu.sync_copy(d_hbm.at[pl.ds(off, DC)], didx)
            pltpu.sync_copy(ones_v, acc_sh.at[didx], add=True)

        plsc.subcore_barrier()
        pltpu.sync_copy(acc_sh.at[pl.ds(s * RPS, RPS)],
                        deg_out.at[pl.ds(c * NP_ + s * RPS, RPS)])

    run = pl.kernel(body, out_type=jax.ShapeDtypeStruct((NC * NP_, D), jnp.float32),
                    mesh=_mesh,
                    scratch_types=[
                        pltpu.VMEM((DC,), jnp.int32),
                        pltpu.VMEM((DC, D), jnp.float32),
                        pltpu.VMEM_SHARED((NP_, D), jnp.float32),
                        pltpu.SemaphoreType.DMA,
                    ])
    return run(dst, zrows, ones_rows)


_BR = 1000


def _tc_combine(acc, degw, x, W_l, W_r, b, relu):
    def body(acc_ref, deg_ref, x_ref, wl_ref, wr_ref, b_ref, o_ref):
        s = acc_ref[0] + acc_ref[1]
        deg = deg_ref[0, :, 0] + deg_ref[1, :, 0]
        mean = s / jnp.clip(deg, 1.0, None)[:, None]
        y = (jnp.dot(mean, wl_ref[...], preferred_element_type=jnp.float32)
             + jnp.dot(x_ref[...], wr_ref[...], preferred_element_type=jnp.float32)
             + b_ref[...])
        o_ref[...] = jnp.maximum(y, 0.0) if relu else y

    return pl.pallas_call(
        body,
        grid=(N // _BR,),
        in_specs=[
            pl.BlockSpec((NC, _BR, D), lambda i: (0, i, 0)),
            pl.BlockSpec((NC, _BR, D), lambda i: (0, i, 0)),
            pl.BlockSpec((_BR, D), lambda i: (i, 0)),
            pl.BlockSpec((D, D), lambda i: (0, 0)),
            pl.BlockSpec((D, D), lambda i: (0, 0)),
            pl.BlockSpec((1, D), lambda i: (0, 0)),
        ],
        out_specs=pl.BlockSpec((_BR, D), lambda i: (i, 0)),
        out_shape=jax.ShapeDtypeStruct((N, D), jnp.float32),
    )(acc, degw, x, W_l, W_r, b)


@jax.jit
def kernel(x, edge_index, W_l1, W_r1, b1, W_l2, W_r2, b2):
    ef = edge_index.astype(jnp.int32).reshape(2 * E)
    zrows = jnp.zeros((RPS, D), jnp.float32)
    ones_rows = jnp.ones((CHUNK, D), jnp.float32)
    b1r = b1.reshape(1, D)
    b2r = b2.reshape(1, D)

    degw = _sc_degree(ef[E:], zrows, ones_rows).reshape(NC, NP_, D)
    acc1 = _sc_aggregate(x, ef, zrows).reshape(NC, NP_, D)
    h = _tc_combine(acc1, degw, x, W_l1, W_r1, b1r, True)
    acc2 = _sc_aggregate(h, ef, zrows).reshape(NC, NP_, D)
    out = _tc_combine(acc2, degw, h, W_l2, W_r2, b2r, False)
    return out


# speedup vs baseline: 9.0491x; 1.2794x over previous
import jax
import jax.numpy as jnp
from jax import lax
from jax.experimental import pallas as pl
from jax.experimental.pallas import tpu as pltpu
from jax.experimental.pallas import tpu_sc as plsc

N = 10000
E = 320000
D = 128

NC = 2
NS = 16
NW = NC * NS

EPC = 128
NCH = E // EPC
CPW = NCH // NW
NTAIL = NCH - CPW * NW
NP_ = 10240
RPS = NP_ // NS

_mesh = plsc.VectorSubcoreMesh(core_axis_name="c", subcore_axis_name="s")


def _sc_aggregate(x, ef, zrows):
    def body(x_hbm, ef_hbm, z_hbm, acc_out,
             sa0, sa1, da0, da1, rows0, rows1, acc_sh, sem):
        c = lax.axis_index("c")
        s = lax.axis_index("s")
        wid = c * NS + s

        pltpu.sync_copy(z_hbm, acc_sh.at[pl.ds(s * RPS, RPS)])
        plsc.subcore_barrier()

        o0 = wid * EPC
        pltpu.sync_copy(ef_hbm.at[pl.ds(o0, EPC)], sa0)
        pltpu.sync_copy(ef_hbm.at[pl.ds(E + o0, EPC)], da0)
        pltpu.async_copy(x_hbm.at[sa0], rows0, sem)

        @pl.loop(0, CPW, step=2)
        def _(ci):
            o1 = (wid + NW * (ci + 1)) * EPC
            pltpu.sync_copy(ef_hbm.at[pl.ds(o1, EPC)], sa1)
            pltpu.sync_copy(ef_hbm.at[pl.ds(E + o1, EPC)], da1)
            pltpu.make_async_copy(x_hbm.at[sa0], rows0, sem).wait()
            pltpu.async_copy(x_hbm.at[sa1], rows1, sem)
            pltpu.sync_copy(rows0, acc_sh.at[da0], add=True)

            @pl.when(ci + 2 < CPW)
            def _():
                o2 = (wid + NW * (ci + 2)) * EPC
                pltpu.sync_copy(ef_hbm.at[pl.ds(o2, EPC)], sa0)
                pltpu.sync_copy(ef_hbm.at[pl.ds(E + o2, EPC)], da0)

            pltpu.make_async_copy(x_hbm.at[sa1], rows1, sem).wait()

            @pl.when(ci + 2 < CPW)
            def _():
                pltpu.async_copy(x_hbm.at[sa0], rows0, sem)

            pltpu.sync_copy(rows1, acc_sh.at[da1], add=True)

        @pl.when(wid < NTAIL)
        def _():
            ot = (NCH - NTAIL + wid) * EPC
            pltpu.sync_copy(ef_hbm.at[pl.ds(ot, EPC)], sa0)
            pltpu.sync_copy(ef_hbm.at[pl.ds(E + ot, EPC)], da0)
            pltpu.async_copy(x_hbm.at[sa0], rows0, sem).wait()
            pltpu.sync_copy(rows0, acc_sh.at[da0], add=True)

        plsc.subcore_barrier()
        pltpu.sync_copy(acc_sh.at[pl.ds(s * RPS, RPS)],
                        acc_out.at[pl.ds(c * NP_ + s * RPS, RPS)])

    run = pl.kernel(body, out_type=jax.ShapeDtypeStruct((NC * NP_, D), jnp.float32),
                    mesh=_mesh,
                    scratch_types=[
                        pltpu.VMEM((EPC,), jnp.int32),
                        pltpu.VMEM((EPC,), jnp.int32),
                        pltpu.VMEM((EPC,), jnp.int32),
                        pltpu.VMEM((EPC,), jnp.int32),
                        pltpu.VMEM((EPC, D), jnp.float32),
                        pltpu.VMEM((EPC, D), jnp.float32),
                        pltpu.VMEM_SHARED((NP_, D), jnp.float32),
                        pltpu.SemaphoreType.DMA,
                    ])
    return run(x, ef, zrows)


def _sc_degree(ef, zrows, ones_rows):
    def body(ef_hbm, z_hbm, ones_hbm, deg_out, didx0, didx1, ones_v, acc_sh, sem):
        c = lax.axis_index("c")
        s = lax.axis_index("s")
        wid = c * NS + s

        pltpu.sync_copy(z_hbm, acc_sh.at[pl.ds(s * RPS, RPS)])
        pltpu.sync_copy(ones_hbm, ones_v)
        plsc.subcore_barrier()

        pltpu.sync_copy(ef_hbm.at[pl.ds(E + wid * EPC, EPC)], didx0)

        @pl.loop(0, CPW, step=2)
        def _(ci):
            d0 = pltpu.async_copy(ones_v, acc_sh.at[didx0], sem, add=True)
            o1 = E + (wid + NW * (ci + 1)) * EPC
            pltpu.sync_copy(ef_hbm.at[pl.ds(o1, EPC)], didx1)
            d0.wait()
            d1 = pltpu.async_copy(ones_v, acc_sh.at[didx1], sem, add=True)

            @pl.when(ci + 2 < CPW)
            def _():
                o2 = E + (wid + NW * (ci + 2)) * EPC
                pltpu.sync_copy(ef_hbm.at[pl.ds(o2, EPC)], didx0)

            d1.wait()

        @pl.when(wid < NTAIL)
        def _():
            ot = E + (NCH - NTAIL + wid) * EPC
            pltpu.sync_copy(ef_hbm.at[pl.ds(ot, EPC)], didx0)
            pltpu.sync_copy(ones_v, acc_sh.at[didx0], add=True)

        plsc.subcore_barrier()
        pltpu.sync_copy(acc_sh.at[pl.ds(s * RPS, RPS)],
                        deg_out.at[pl.ds(c * NP_ + s * RPS, RPS)])

    run = pl.kernel(body, out_type=jax.ShapeDtypeStruct((NC * NP_, D), jnp.float32),
                    mesh=_mesh,
                    scratch_types=[
                        pltpu.VMEM((EPC,), jnp.int32),
                        pltpu.VMEM((EPC,), jnp.int32),
                        pltpu.VMEM((EPC, D), jnp.float32),
                        pltpu.VMEM_SHARED((NP_, D), jnp.float32),
                        pltpu.SemaphoreType.DMA,
                    ])
    return run(ef, zrows, ones_rows)


_BR = 1000


def _tc_combine(acc, degw, x, W_l, W_r, b, relu):
    def body(acc_ref, deg_ref, x_ref, wl_ref, wr_ref, b_ref, o_ref):
        s = acc_ref[0] + acc_ref[1]
        deg = deg_ref[0, :, 0] + deg_ref[1, :, 0]
        mean = s / jnp.clip(deg, 1.0, None)[:, None]
        y = (jnp.dot(mean, wl_ref[...], preferred_element_type=jnp.float32)
             + jnp.dot(x_ref[...], wr_ref[...], preferred_element_type=jnp.float32)
             + b_ref[...])
        o_ref[...] = jnp.maximum(y, 0.0) if relu else y

    return pl.pallas_call(
        body,
        grid=(N // _BR,),
        in_specs=[
            pl.BlockSpec((NC, _BR, D), lambda i: (0, i, 0)),
            pl.BlockSpec((NC, _BR, D), lambda i: (0, i, 0)),
            pl.BlockSpec((_BR, D), lambda i: (i, 0)),
            pl.BlockSpec((D, D), lambda i: (0, 0)),
            pl.BlockSpec((D, D), lambda i: (0, 0)),
            pl.BlockSpec((1, D), lambda i: (0, 0)),
        ],
        out_specs=pl.BlockSpec((_BR, D), lambda i: (i, 0)),
        out_shape=jax.ShapeDtypeStruct((N, D), jnp.float32),
    )(acc, degw, x, W_l, W_r, b)


@jax.jit
def kernel(x, edge_index, W_l1, W_r1, b1, W_l2, W_r2, b2):
    ef = edge_index.astype(jnp.int32).reshape(2 * E)
    zrows = jnp.zeros((RPS, D), jnp.float32)
    ones_rows = jnp.ones((EPC, D), jnp.float32)
    b1r = b1.reshape(1, D)
    b2r = b2.reshape(1, D)

    degw = _sc_degree(ef, zrows, ones_rows).reshape(NC, NP_, D)
    acc1 = _sc_aggregate(x, ef, zrows).reshape(NC, NP_, D)
    h = _tc_combine(acc1, degw, x, W_l1, W_r1, b1r, True)
    acc2 = _sc_aggregate(h, ef, zrows).reshape(NC, NP_, D)
    out = _tc_combine(acc2, degw, h, W_l2, W_r2, b2r, False)
    return out
